# Initial kernel scaffold; baseline (speedup 1.0000x reference)
import jax
import jax.numpy as jnp
from jax.experimental import pallas as pl


def kernel(x, expert_ids, W, b, lora_a, lora_b):
    raise NotImplementedError("write your pallas kernel here")


# speedup vs baseline: 4.7904x; 4.7904x over previous
import functools

import jax
import jax.numpy as jnp
from jax.experimental import pallas as pl
from jax.experimental.pallas import tpu as pltpu

_ALPHA = 32.0


def _body(eids_ref, x_ref, wt_ref, la_ref, lbt_ref, b_ref, o_ref, inter_ref,
          *, bt: int, rank: int, er: int, scaling: float):
    j = pl.program_id(0)
    i = pl.program_id(1)

    @pl.when(j == 0)
    def _compute_inter():
        inter = jnp.dot(x_ref[...], la_ref[...],
                        preferred_element_type=jnp.float32)
        eids = eids_ref[0, 0, :]
        col_expert = jax.lax.broadcasted_iota(jnp.int32, (bt, er), 1) // rank
        mask = (col_expert == eids[:, None]).astype(inter.dtype)
        inter_ref[pl.ds(i * bt, bt), :] = inter * mask

    base = jnp.dot(x_ref[...], wt_ref[...], preferred_element_type=jnp.float32)
    delta = jnp.dot(inter_ref[pl.ds(i * bt, bt), :], lbt_ref[...],
                    preferred_element_type=jnp.float32)
    o_ref[...] = base + b_ref[...] + delta * scaling


def kernel(x, expert_ids, W, b, lora_a, lora_b):
    num_tokens, d_in = x.shape
    d_out = W.shape[0]
    num_experts, rank, _ = lora_a.shape
    er = num_experts * rank
    scaling = _ALPHA / float(rank)

    bt = 256
    bo = 512
    nt = num_tokens // bt
    no = d_out // bo

    wt = W.T
    la = lora_a.reshape(er, d_in).T
    lbt = lora_b.transpose(0, 2, 1).reshape(er, d_out)
    b2 = b.reshape(1, d_out)
    eids3 = expert_ids.astype(jnp.int32).reshape(nt, 1, bt)

    grid = (no, nt)
    out = pl.pallas_call(
        functools.partial(_body, bt=bt, rank=rank, er=er, scaling=scaling),
        grid=grid,
        in_specs=[
            pl.BlockSpec((1, 1, bt), lambda j, i: (i, 0, 0)),
            pl.BlockSpec((bt, d_in), lambda j, i: (i, 0)),
            pl.BlockSpec((d_in, bo), lambda j, i: (0, j)),
            pl.BlockSpec((d_in, er), lambda j, i: (0, 0)),
            pl.BlockSpec((er, bo), lambda j, i: (0, j)),
            pl.BlockSpec((1, bo), lambda j, i: (0, j)),
        ],
        out_specs=pl.BlockSpec((bt, bo), lambda j, i: (i, j)),
        out_shape=jax.ShapeDtypeStruct((num_tokens, d_out), jnp.float32),
        scratch_shapes=[pltpu.VMEM((num_tokens, er), jnp.float32)],
        compiler_params=pltpu.CompilerParams(
            dimension_semantics=("arbitrary", "arbitrary"),
        ),
    )(eids3, x, wt, la, lbt, b2)
    return out


# speedup vs baseline: 4.9507x; 1.0335x over previous
import functools

import jax
import jax.numpy as jnp
from jax.experimental import pallas as pl
from jax.experimental.pallas import tpu as pltpu

_ALPHA = 32.0


def _body(eids_ref, x_ref, wt_ref, la_ref, lbt_ref, b_ref, o_ref, inter_ref,
          *, bt: int, rank: int, er: int, scaling: float):
    j = pl.program_id(0)
    i = pl.program_id(1)

    @pl.when(j == 0)
    def _compute_inter():
        inter = jnp.dot(x_ref[...], la_ref[...],
                        preferred_element_type=jnp.float32)
        eids = eids_ref[0, 0, :]
        col_expert = jax.lax.broadcasted_iota(jnp.int32, (bt, er), 1) // rank
        mask = (col_expert == eids[:, None]).astype(inter.dtype)
        inter_ref[pl.ds(i * bt, bt), :] = (inter * mask).astype(inter_ref.dtype)

    base = jnp.dot(x_ref[...], wt_ref[...], preferred_element_type=jnp.float32)
    delta = jnp.dot(inter_ref[pl.ds(i * bt, bt), :], lbt_ref[...],
                    preferred_element_type=jnp.float32)
    o_ref[...] = base + b_ref[...] + delta * scaling


def kernel(x, expert_ids, W, b, lora_a, lora_b):
    num_tokens, d_in = x.shape
    d_out = W.shape[0]
    num_experts, rank, _ = lora_a.shape
    er = num_experts * rank
    scaling = _ALPHA / float(rank)

    bt = 256
    bo = 512
    nt = num_tokens // bt
    no = d_out // bo

    cdt = jnp.bfloat16
    xc = x.astype(cdt)
    wt = W.T.astype(cdt)
    la = lora_a.reshape(er, d_in).T.astype(cdt)
    lbt = (lora_b.transpose(0, 2, 1)
           .reshape(er, d_out).astype(cdt))
    b2 = b.reshape(1, d_out)
    eids3 = expert_ids.astype(jnp.int32).reshape(nt, 1, bt)

    grid = (no, nt)
    out = pl.pallas_call(
        functools.partial(_body, bt=bt, rank=rank, er=er, scaling=scaling),
        grid=grid,
        in_specs=[
            pl.BlockSpec((1, 1, bt), lambda j, i: (i, 0, 0)),
            pl.BlockSpec((bt, d_in), lambda j, i: (i, 0)),
            pl.BlockSpec((d_in, bo), lambda j, i: (0, j)),
            pl.BlockSpec((d_in, er), lambda j, i: (0, 0)),
            pl.BlockSpec((er, bo), lambda j, i: (0, j)),
            pl.BlockSpec((1, bo), lambda j, i: (0, j)),
        ],
        out_specs=pl.BlockSpec((bt, bo), lambda j, i: (i, j)),
        out_shape=jax.ShapeDtypeStruct((num_tokens, d_out), jnp.float32),
        scratch_shapes=[pltpu.VMEM((num_tokens, er), cdt)],
        compiler_params=pltpu.CompilerParams(
            dimension_semantics=("arbitrary", "arbitrary"),
        ),
    )(eids3, xc, wt, la, lbt, b2)
    return out


# speedup vs baseline: 6.9493x; 1.4037x over previous
import functools

import jax
import jax.numpy as jnp
from jax.experimental import pallas as pl
from jax.experimental.pallas import tpu as pltpu

_ALPHA = 32.0


def _body(eids_ref, x_ref, wt_ref, la_ref, lbt_ref, b_ref, o_ref, inter_ref,
          *, bt: int, rank: int, er: int, scaling: float):
    j = pl.program_id(0)
    i = pl.program_id(1)

    @pl.when(j == 0)
    def _compute_inter():
        inter = jnp.dot(x_ref[...], la_ref[...],
                        preferred_element_type=jnp.float32)
        eids = eids_ref[0, 0, :]
        col_expert = jax.lax.broadcasted_iota(jnp.int32, (bt, er), 1) // rank
        mask = (col_expert == eids[:, None]).astype(inter.dtype)
        inter_ref[pl.ds(i * bt, bt), :] = (inter * mask).astype(inter_ref.dtype)

    base = jnp.dot(x_ref[...], wt_ref[...], preferred_element_type=jnp.float32)
    delta = jnp.dot(inter_ref[pl.ds(i * bt, bt), :], lbt_ref[...],
                    preferred_element_type=jnp.float32)
    o_ref[...] = base + b_ref[...] + delta * scaling


def kernel(x, expert_ids, W, b, lora_a, lora_b):
    num_tokens, d_in = x.shape
    d_out = W.shape[0]
    num_experts, rank, _ = lora_a.shape
    er = num_experts * rank
    scaling = _ALPHA / float(rank)

    bt = 512
    bo = 1024
    nt = num_tokens // bt
    no = d_out // bo

    cdt = jnp.bfloat16
    xc = x.astype(cdt)
    wt = W.T.astype(cdt)
    la = lora_a.reshape(er, d_in).T.astype(cdt)
    lbt = (lora_b.transpose(0, 2, 1)
           .reshape(er, d_out).astype(cdt))
    b2 = b.reshape(1, d_out)
    eids3 = expert_ids.astype(jnp.int32).reshape(nt, 1, bt)

    grid = (no, nt)
    out = pl.pallas_call(
        functools.partial(_body, bt=bt, rank=rank, er=er, scaling=scaling),
        grid=grid,
        in_specs=[
            pl.BlockSpec((1, 1, bt), lambda j, i: (i, 0, 0)),
            pl.BlockSpec((bt, d_in), lambda j, i: (i, 0)),
            pl.BlockSpec((d_in, bo), lambda j, i: (0, j)),
            pl.BlockSpec((d_in, er), lambda j, i: (0, 0)),
            pl.BlockSpec((er, bo), lambda j, i: (0, j)),
            pl.BlockSpec((1, bo), lambda j, i: (0, j)),
        ],
        out_specs=pl.BlockSpec((bt, bo), lambda j, i: (i, j)),
        out_shape=jax.ShapeDtypeStruct((num_tokens, d_out), jnp.float32),
        scratch_shapes=[pltpu.VMEM((num_tokens, er), cdt)],
        compiler_params=pltpu.CompilerParams(
            dimension_semantics=("arbitrary", "arbitrary"),
        ),
    )(eids3, xc, wt, la, lbt, b2)
    return out
